# Initial kernel scaffold; baseline (speedup 1.0000x reference)
import jax
import jax.numpy as jnp
from jax.experimental import pallas as pl


def kernel(x, edge_index, W1l, b1, W1r, W2l, b2, W2r):
    raise NotImplementedError("write your pallas kernel here")


# speedup vs baseline: 6.2292x; 6.2292x over previous
import functools

import jax
import jax.numpy as jnp
from jax import lax
from jax.experimental import pallas as pl
from jax.experimental.pallas import tpu as pltpu
from jax.experimental.pallas import tpu_sc as plsc

N = 10000
E = 320000
D_FEAT = 128
D_HID = 256
D_OUT = 128

NC = 2
NS = 16
K = 80
NB = E // NS // K
ZROWS = 80
NCH = N // ZROWS
TPC = -(-NCH // NS)
DGC = 16
DGR = N // DGC


def _zero_vmem(ref, nrows, ncols16):
    z = jnp.zeros((16,), jnp.float32)

    def body(i, carry):
        for j in range(ncols16):
            ref[i, pl.ds(j * 16, 16)] = z
        return carry

    lax.fori_loop(0, nrows, body, 0)


def _make_sc_pass(width, with_deg):
    mesh = plsc.VectorSubcoreMesh(core_axis_name="c", subcore_axis_name="s")
    agg_ty = jax.ShapeDtypeStruct((NC, N, width), jnp.float32)
    deg_ty = jax.ShapeDtypeStruct((NS, DGR, DGC), jnp.float32)
    out_type = (agg_ty, deg_ty) if with_deg else agg_ty
    scratch = [
        pltpu.VMEM((NB, K), jnp.int32),
        pltpu.VMEM((NB, K), jnp.int32),
        pltpu.VMEM((K, width), jnp.float32),
        pltpu.VMEM((K, width), jnp.float32),
        pltpu.VMEM((ZROWS, width), jnp.float32),
    ]
    if with_deg:
        scratch.append(pltpu.VMEM((DGR, DGC), jnp.float32))
    scratch += [
        pltpu.VMEM_SHARED((N, width), jnp.float32),
        pltpu.SemaphoreType.DMA,
        pltpu.SemaphoreType.DMA,
    ]

    @functools.partial(pl.kernel, out_type=out_type, mesh=mesh,
                       scratch_types=scratch,
                       compiler_params=pltpu.CompilerParams(
                           use_tc_tiling_on_sc=False,
                           needs_layout_passes=False))
    def sc_pass(src_hbm, esrc, edst, *refs):
        if with_deg:
            (aggp, degp, srcv, dstv, rows0, rows1, zf, degref, acc,
             sem0, sem1) = refs
        else:
            aggp, srcv, dstv, rows0, rows1, zf, acc, sem0, sem1 = refs
            degp = degref = None

        c = lax.axis_index("c")
        s = lax.axis_index("s")

        pltpu.sync_copy(esrc.at[c, s], srcv)
        pltpu.sync_copy(edst.at[s], dstv)

        _zero_vmem(zf, ZROWS, width // 16)
        if with_deg:
            _zero_vmem(degref, DGR, 1)
        for t in range(TPC):
            ch = s + t * NS

            @pl.when(ch < NCH)
            def _():
                pltpu.sync_copy(zf, acc.at[pl.ds(ch * ZROWS, ZROWS)])
        plsc.subcore_barrier()

        ones16 = jnp.ones((16,), jnp.float32)

        def start(j, rbuf, sem):
            pltpu.async_copy(src_hbm.at[srcv.at[j]], rbuf, sem)

        def finish(j, rbuf, sem):
            pltpu.make_async_copy(src_hbm.at[srcv.at[j]], rbuf, sem).wait()
            pltpu.sync_copy(rbuf, acc.at[dstv.at[j]], add=True)
            if with_deg:
                @pl.when(c == 0)
                def _():
                    v4 = jnp.full((16,), 4, jnp.int32)
                    v15 = jnp.full((16,), 15, jnp.int32)
                    for k in range(K // 16):
                        dv = dstv[j, pl.ds(k * 16, 16)]
                        r = lax.shift_right_logical(dv, v4)
                        cc = lax.bitwise_and(dv, v15)
                        plsc.addupdate_scatter(degref, [r, cc], ones16)

        start(0, rows0, sem0)

        def body(i, carry):
            j = 2 * i
            start(j + 1, rows1, sem1)
            finish(j, rows0, sem0)

            @pl.when(j + 2 < NB)
            def _():
                start(j + 2, rows0, sem0)

            finish(j + 1, rows1, sem1)
            return carry

        lax.fori_loop(0, NB // 2, body, 0)

        plsc.subcore_barrier()
        for t in range(TPC):
            ch = s + t * NS

            @pl.when(ch < NCH)
            def _():
                r0 = ch * ZROWS
                pltpu.sync_copy(acc.at[pl.ds(r0, ZROWS)],
                                aggp.at[c, pl.ds(r0, ZROWS)])
        if with_deg:
            @pl.when(c == 0)
            def _():
                pltpu.sync_copy(degref, degp.at[s])

    return sc_pass


_BM = 400


def _recip_deg(degp_ref):
    degsum = jnp.sum(degp_ref[...], axis=0)
    return 1.0 / jnp.maximum(degsum, 1.0)


def _tc1_body(x_ref, aggp_ref, degp_ref, w1lt_ref, w1rt_ref, b1_ref,
              hcat_ref):
    recip = _recip_deg(degp_ref)
    agg = jnp.concatenate([aggp_ref[0], aggp_ref[1]], axis=1) * recip
    h = (jnp.dot(agg, w1lt_ref[...], preferred_element_type=jnp.float32)
         + b1_ref[...]
         + jnp.dot(x_ref[...], w1rt_ref[...],
                   preferred_element_type=jnp.float32))
    h = jnp.maximum(h, 0.0)
    for q in range(4):
        hcat_ref[q] = h[:, q * 64:(q + 1) * 64]


def _tc2_body(hcat_ref, a2pa_ref, a2pb_ref, degp_ref, w2lt_ref, w2rt_ref,
              b2_ref, out_ref):
    recip = _recip_deg(degp_ref)
    agg = jnp.concatenate(
        [a2pa_ref[0], a2pa_ref[1], a2pb_ref[0], a2pb_ref[1]], axis=1) * recip
    h = jnp.concatenate(
        [hcat_ref[0], hcat_ref[1], hcat_ref[2], hcat_ref[3]], axis=1)
    out_ref[...] = (
        jnp.dot(agg, w2lt_ref[...], preferred_element_type=jnp.float32)
        + b2_ref[...]
        + jnp.dot(h, w2rt_ref[...], preferred_element_type=jnp.float32))


def _full(shape):
    return pl.BlockSpec(shape, lambda i: (0,) * len(shape))


_DEG_SPEC = pl.BlockSpec((NS, _BM, 1), lambda i: (0, i, 0))


def _tc1(x, aggp, degp, w1lt, w1rt, b1):
    return pl.pallas_call(
        _tc1_body,
        grid=(N // _BM,),
        in_specs=[
            pl.BlockSpec((_BM, D_FEAT), lambda i: (i, 0)),
            pl.BlockSpec((NC, _BM, D_FEAT // 2), lambda i: (0, i, 0)),
            _DEG_SPEC,
            _full((D_FEAT, D_HID)),
            _full((D_FEAT, D_HID)),
            _full((1, D_HID)),
        ],
        out_specs=pl.BlockSpec((4, _BM, 64), lambda i: (0, i, 0)),
        out_shape=jax.ShapeDtypeStruct((4, N, 64), jnp.float32),
    )(x, aggp, degp, w1lt, w1rt, b1)


def _tc2(hcat, a2pa, a2pb, degp, w2lt, w2rt, b2):
    return pl.pallas_call(
        _tc2_body,
        grid=(N // _BM,),
        in_specs=[
            pl.BlockSpec((4, _BM, 64), lambda i: (0, i, 0)),
            pl.BlockSpec((NC, _BM, 64), lambda i: (0, i, 0)),
            pl.BlockSpec((NC, _BM, 64), lambda i: (0, i, 0)),
            _DEG_SPEC,
            _full((D_HID, D_OUT)),
            _full((D_HID, D_OUT)),
            _full((1, D_OUT)),
        ],
        out_specs=pl.BlockSpec((_BM, D_OUT), lambda i: (i, 0)),
        out_shape=jax.ShapeDtypeStruct((N, D_OUT), jnp.float32),
    )(hcat, a2pa, a2pb, degp, w2lt, w2rt, b2)


_sc_pass1 = _make_sc_pass(D_FEAT // 2, with_deg=True)
_sc_pass2 = _make_sc_pass(D_FEAT // 2, with_deg=False)


def kernel(x, edge_index, W1l, b1, W1r, W2l, b2, W2r):
    src = edge_index[0].astype(jnp.int32)
    dst = edge_index[1].astype(jnp.int32)

    esrc = jnp.concatenate([src, src + N]).reshape(NC, NS, NB, K)
    edst = dst.reshape(NS, NB, K)
    xcat = jnp.concatenate([x[:, :D_FEAT // 2], x[:, D_FEAT // 2:]], axis=0)

    aggp, degp = _sc_pass1(xcat, esrc, edst)
    degp = degp.reshape(NS, N, 1)
    hcat = _tc1(x, aggp, degp, W1l.T, W1r.T, b1.reshape(1, D_HID))
    a2pa = _sc_pass2(hcat[:2].reshape(NC * N, 64), esrc, edst)
    a2pb = _sc_pass2(hcat[2:].reshape(NC * N, 64), esrc, edst)
    out = _tc2(hcat, a2pa, a2pb, degp, W2l.T, W2r.T, b2.reshape(1, D_OUT))
    return out
